# baseline (speedup 1.0000x reference)
import functools

import jax
import jax.numpy as jnp
from jax import lax
from jax.experimental import pallas as pl
from jax.experimental.pallas import tpu as pltpu
from jax.experimental.pallas import tpu_sc as plsc

B = 4
S = 4096
D = 2048
K = 512
SBLK = 256
RBLK = 256


def _norm_kernel(x_ref, imp_ref, hm_ref):
    x = x_ref[...]
    c0 = x[:, :, 0:128]
    acc = c0 * c0
    for c in range(1, 16):
        xc = x[:, :, c * 128:(c + 1) * 128]
        acc = acc + xc * xc
    s2 = acc
    for j in range(1, 16):
        s2 = s2 + pltpu.roll(acc, 128 - 8 * j, axis=2)
    t1 = s2 + pltpu.roll(s2, 124, axis=2)
    t2 = t1 + pltpu.roll(t1, 126, axis=2)
    t3 = t2 + pltpu.roll(t2, 127, axis=2)
    ss = t3[:, :, 0]
    n = jnp.sqrt(ss)
    imp_ref[...] = jnp.mean(n, axis=0)
    hm_ref[...] = jnp.mean(x, axis=0)


def _norm_pass(hidden_states):
    return pl.pallas_call(
        _norm_kernel,
        grid=(S // SBLK,),
        in_specs=[pl.BlockSpec((B, SBLK, D), lambda i: (0, i, 0))],
        out_specs=[pl.BlockSpec((SBLK,), lambda i: (i,)),
                   pl.BlockSpec((SBLK, D), lambda i: (i, 0))],
        out_shape=[jax.ShapeDtypeStruct((S,), jnp.float32),
                   jax.ShapeDtypeStruct((S, D), jnp.float32)],
    )(hidden_states)


def _rank_kernel(imp_full_ref, imp_blk_ref, rank_ref):
    i = pl.program_id(0)
    kt = imp_full_ref[...]
    ks = imp_blk_ref[...]
    ktr = kt[None, :]
    ksc = ks[:, None]
    gt = (ktr > ksc).astype(jnp.int32)
    it = lax.broadcasted_iota(jnp.int32, (RBLK, S), 1)
    isc = i * RBLK + lax.broadcasted_iota(jnp.int32, (RBLK, S), 0)
    tie = ((ktr == ksc) & (it < isc)).astype(jnp.int32)
    rank_ref[...] = jnp.sum(gt + tie, axis=1)


def _rank_pass(imp):
    return pl.pallas_call(
        _rank_kernel,
        grid=(S // RBLK,),
        in_specs=[pl.BlockSpec((S,), lambda i: (0,)),
                  pl.BlockSpec((RBLK,), lambda i: (i,))],
        out_specs=pl.BlockSpec((RBLK,), lambda i: (i,)),
        out_shape=jax.ShapeDtypeStruct((S,), jnp.int32),
    )(imp, imp)


IBLK = 256


def _invert_kernel(rank_ref, idx_ref):
    i = pl.program_id(0)
    r = rank_ref[...]
    rr = r[None, :]
    rows = i * IBLK + lax.broadcasted_iota(jnp.int32, (IBLK, S), 0)
    it = lax.broadcasted_iota(jnp.int32, (IBLK, S), 1)
    sel = jnp.where(rr == rows, it, 0)
    idx_ref[...] = jnp.sum(sel, axis=1)


def _invert_pass(rank):
    return pl.pallas_call(
        _invert_kernel,
        grid=(K // IBLK,),
        in_specs=[pl.BlockSpec((S,), lambda i: (0,))],
        out_specs=pl.BlockSpec((IBLK,), lambda i: (i,)),
        out_shape=jax.ShapeDtypeStruct((K,), jnp.int32),
    )(rank)


def _make_sc_gather():
    info = plsc.get_sparse_core_info()
    nc, ns = info.num_cores, info.num_subcores
    nw = nc * ns
    b_per_w = K // nw
    mesh = plsc.VectorSubcoreMesh(core_axis_name="c", subcore_axis_name="s")

    @functools.partial(
        pl.kernel, mesh=mesh,
        out_type=jax.ShapeDtypeStruct((K, D), jnp.float32),
        scratch_types=[
            pltpu.VMEM((b_per_w,), jnp.int32),
            pltpu.VMEM((b_per_w, D), jnp.float32),
            pltpu.SemaphoreType.DMA,
        ],
    )
    def sc_gather(hmean_hbm, idx_hbm, out_hbm, idx_v, rows_v, sem):
        wid = lax.axis_index("s") * nc + lax.axis_index("c")
        base = wid * b_per_w
        pltpu.sync_copy(idx_hbm.at[pl.ds(base, b_per_w)], idx_v)
        pltpu.async_copy(hmean_hbm.at[idx_v], rows_v, sem).wait()
        pltpu.sync_copy(rows_v, out_hbm.at[pl.ds(base, b_per_w)])

    return sc_gather


def kernel(hidden_states, memory):
    importance, hmean = _norm_pass(hidden_states)
    rank = _rank_pass(importance)
    topk_indices = _invert_pass(rank)
    sc = _make_sc_gather()
    return sc(hmean, topk_indices)


# speedup vs baseline: 1.0010x; 1.0010x over previous
import functools

import jax
import jax.numpy as jnp
from jax import lax
from jax.experimental import pallas as pl
from jax.experimental.pallas import tpu as pltpu
from jax.experimental.pallas import tpu_sc as plsc

B = 4
S = 4096
D = 2048
K = 512
SBLK = 256
RBLK = 256


N_NORM = S // SBLK
N_RANK = S // RBLK
N_INV = 2
IBLK2 = K // N_INV


def _fused_kernel(x_ref, hm_ref, idx_ref, imp_s, rank_s):
    i = pl.program_id(0)

    @pl.when(i < N_NORM)
    def _norm_phase():
        x = x_ref[...]
        c0 = x[:, :, 0:128]
        acc = c0 * c0
        for c in range(1, 16):
            xc = x[:, :, c * 128:(c + 1) * 128]
            acc = acc + xc * xc
        s2 = acc
        for j in range(1, 16):
            s2 = s2 + pltpu.roll(acc, 128 - 8 * j, axis=2)
        t1 = s2 + pltpu.roll(s2, 124, axis=2)
        t2 = t1 + pltpu.roll(t1, 126, axis=2)
        t3 = t2 + pltpu.roll(t2, 127, axis=2)
        ss = t3[:, :, 0]
        n = jnp.sqrt(ss)
        imp_s[pl.ds(pl.multiple_of(i * SBLK, 128), SBLK)] = jnp.mean(n, axis=0)
        hm_ref[...] = jnp.mean(x, axis=0)

    @pl.when((i >= N_NORM) & (i < N_NORM + N_RANK))
    def _rank_phase():
        r = i - N_NORM
        kt = imp_s[...]
        ks = imp_s[pl.ds(pl.multiple_of(r * RBLK, 128), RBLK)]
        ktr = kt[None, :]
        ksc = ks[:, None]
        gt = (ktr > ksc).astype(jnp.int32)
        it = lax.broadcasted_iota(jnp.int32, (RBLK, S), 1)
        isc = r * RBLK + lax.broadcasted_iota(jnp.int32, (RBLK, S), 0)
        tie = ((ktr == ksc) & (it < isc)).astype(jnp.int32)
        rank_s[pl.ds(pl.multiple_of(r * RBLK, 128), RBLK)] = jnp.sum(
            gt + tie, axis=1)

    @pl.when(i >= N_NORM + N_RANK)
    def _invert_phase():
        v = i - N_NORM - N_RANK
        rr = rank_s[...][None, :]
        rows = v * IBLK2 + lax.broadcasted_iota(jnp.int32, (IBLK2, S), 0)
        it = lax.broadcasted_iota(jnp.int32, (IBLK2, S), 1)
        sel = jnp.where(rr == rows, it, 0)
        idx_ref[...] = jnp.sum(sel, axis=1)


def _fused_pass(hidden_states):
    nsteps = N_NORM + N_RANK + N_INV
    hmean, idx = pl.pallas_call(
        _fused_kernel,
        grid=(nsteps,),
        in_specs=[pl.BlockSpec(
            (B, SBLK, D),
            lambda i: (0, jnp.minimum(i, N_NORM - 1), 0))],
        out_specs=[
            pl.BlockSpec((SBLK, D), lambda i: (jnp.minimum(i, N_NORM - 1), 0)),
            pl.BlockSpec((IBLK2,),
                         lambda i: (jnp.maximum(i - (N_NORM + N_RANK), 0),)),
        ],
        out_shape=[jax.ShapeDtypeStruct((S, D), jnp.float32),
                   jax.ShapeDtypeStruct((K,), jnp.int32)],
        scratch_shapes=[pltpu.VMEM((S,), jnp.float32),
                        pltpu.VMEM((S,), jnp.int32)],
    )(hidden_states)
    return hmean, idx


def _norm_kernel(x_ref, imp_ref, hm_ref):
    x = x_ref[...]
    c0 = x[:, :, 0:128]
    acc = c0 * c0
    for c in range(1, 16):
        xc = x[:, :, c * 128:(c + 1) * 128]
        acc = acc + xc * xc
    s2 = acc
    for j in range(1, 16):
        s2 = s2 + pltpu.roll(acc, 128 - 8 * j, axis=2)
    t1 = s2 + pltpu.roll(s2, 124, axis=2)
    t2 = t1 + pltpu.roll(t1, 126, axis=2)
    t3 = t2 + pltpu.roll(t2, 127, axis=2)
    ss = t3[:, :, 0]
    n = jnp.sqrt(ss)
    imp_ref[...] = jnp.mean(n, axis=0)
    hm_ref[...] = jnp.mean(x, axis=0)


def _norm_pass(hidden_states):
    return pl.pallas_call(
        _norm_kernel,
        grid=(S // SBLK,),
        in_specs=[pl.BlockSpec((B, SBLK, D), lambda i: (0, i, 0))],
        out_specs=[pl.BlockSpec((SBLK,), lambda i: (i,)),
                   pl.BlockSpec((SBLK, D), lambda i: (i, 0))],
        out_shape=[jax.ShapeDtypeStruct((S,), jnp.float32),
                   jax.ShapeDtypeStruct((S, D), jnp.float32)],
    )(hidden_states)


def _rank_kernel(imp_full_ref, imp_blk_ref, rank_ref):
    i = pl.program_id(0)
    kt = imp_full_ref[...]
    ks = imp_blk_ref[...]
    ktr = kt[None, :]
    ksc = ks[:, None]
    gt = (ktr > ksc).astype(jnp.int32)
    it = lax.broadcasted_iota(jnp.int32, (RBLK, S), 1)
    isc = i * RBLK + lax.broadcasted_iota(jnp.int32, (RBLK, S), 0)
    tie = ((ktr == ksc) & (it < isc)).astype(jnp.int32)
    rank_ref[...] = jnp.sum(gt + tie, axis=1)


def _rank_pass(imp):
    return pl.pallas_call(
        _rank_kernel,
        grid=(S // RBLK,),
        in_specs=[pl.BlockSpec((S,), lambda i: (0,)),
                  pl.BlockSpec((RBLK,), lambda i: (i,))],
        out_specs=pl.BlockSpec((RBLK,), lambda i: (i,)),
        out_shape=jax.ShapeDtypeStruct((S,), jnp.int32),
    )(imp, imp)


IBLK = 256


def _invert_kernel(rank_ref, idx_ref):
    i = pl.program_id(0)
    r = rank_ref[...]
    rr = r[None, :]
    rows = i * IBLK + lax.broadcasted_iota(jnp.int32, (IBLK, S), 0)
    it = lax.broadcasted_iota(jnp.int32, (IBLK, S), 1)
    sel = jnp.where(rr == rows, it, 0)
    idx_ref[...] = jnp.sum(sel, axis=1)


def _invert_pass(rank):
    return pl.pallas_call(
        _invert_kernel,
        grid=(K // IBLK,),
        in_specs=[pl.BlockSpec((S,), lambda i: (0,))],
        out_specs=pl.BlockSpec((IBLK,), lambda i: (i,)),
        out_shape=jax.ShapeDtypeStruct((K,), jnp.int32),
    )(rank)


def _make_sc_gather():
    info = plsc.get_sparse_core_info()
    nc, ns = info.num_cores, info.num_subcores
    nw = nc * ns
    b_per_w = K // nw
    mesh = plsc.VectorSubcoreMesh(core_axis_name="c", subcore_axis_name="s")

    @functools.partial(
        pl.kernel, mesh=mesh,
        out_type=jax.ShapeDtypeStruct((K, D), jnp.float32),
        scratch_types=[
            pltpu.VMEM((b_per_w,), jnp.int32),
            pltpu.VMEM((b_per_w, D), jnp.float32),
            pltpu.SemaphoreType.DMA,
        ],
    )
    def sc_gather(hmean_hbm, idx_hbm, out_hbm, idx_v, rows_v, sem):
        wid = lax.axis_index("s") * nc + lax.axis_index("c")
        base = wid * b_per_w
        pltpu.sync_copy(idx_hbm.at[pl.ds(base, b_per_w)], idx_v)
        pltpu.async_copy(hmean_hbm.at[idx_v], rows_v, sem).wait()
        pltpu.sync_copy(rows_v, out_hbm.at[pl.ds(base, b_per_w)])

    return sc_gather


def kernel(hidden_states, memory):
    hmean, topk_indices = _fused_pass(hidden_states)
    sc = _make_sc_gather()
    return sc(hmean, topk_indices)


# speedup vs baseline: 1.0133x; 1.0123x over previous
import functools

import jax
import jax.numpy as jnp
from jax import lax
from jax.experimental import pallas as pl
from jax.experimental.pallas import tpu as pltpu
from jax.experimental.pallas import tpu_sc as plsc

B = 4
S = 4096
D = 2048
K = 512
SBLK = 512
RBLK = 512


N_NORM = S // SBLK
N_RANK = S // RBLK
N_INV = 2
IBLK2 = K // N_INV


def _fused_kernel(x_ref, hm_ref, idx_ref, imp_s, rank_s):
    i = pl.program_id(0)

    @pl.when(i < N_NORM)
    def _norm_phase():
        x = x_ref[...]
        c0 = x[:, :, 0:128]
        acc = c0 * c0
        for c in range(1, 16):
            xc = x[:, :, c * 128:(c + 1) * 128]
            acc = acc + xc * xc
        s2 = acc
        for j in range(1, 16):
            s2 = s2 + pltpu.roll(acc, 128 - 8 * j, axis=2)
        t1 = s2 + pltpu.roll(s2, 124, axis=2)
        t2 = t1 + pltpu.roll(t1, 126, axis=2)
        t3 = t2 + pltpu.roll(t2, 127, axis=2)
        ss = t3[:, :, 0]
        n = jnp.sqrt(ss)
        imp_s[pl.ds(pl.multiple_of(i * SBLK, 128), SBLK)] = jnp.mean(n, axis=0)
        hm_ref[...] = jnp.mean(x, axis=0)

    @pl.when((i >= N_NORM) & (i < N_NORM + N_RANK))
    def _rank_phase():
        r = i - N_NORM
        kt = imp_s[...]
        ks = imp_s[pl.ds(pl.multiple_of(r * RBLK, 128), RBLK)]
        ktr = kt[None, :]
        ksc = ks[:, None]
        gt = (ktr > ksc).astype(jnp.int32)
        it = lax.broadcasted_iota(jnp.int32, (RBLK, S), 1)
        isc = r * RBLK + lax.broadcasted_iota(jnp.int32, (RBLK, S), 0)
        tie = ((ktr == ksc) & (it < isc)).astype(jnp.int32)
        rank_s[pl.ds(pl.multiple_of(r * RBLK, 128), RBLK)] = jnp.sum(
            gt + tie, axis=1)

    @pl.when(i >= N_NORM + N_RANK)
    def _invert_phase():
        v = i - N_NORM - N_RANK
        rr = rank_s[...][None, :]
        rows = v * IBLK2 + lax.broadcasted_iota(jnp.int32, (IBLK2, S), 0)
        it = lax.broadcasted_iota(jnp.int32, (IBLK2, S), 1)
        sel = jnp.where(rr == rows, it, 0)
        idx_ref[...] = jnp.sum(sel, axis=1)


def _fused_pass(hidden_states):
    nsteps = N_NORM + N_RANK + N_INV
    hmean, idx = pl.pallas_call(
        _fused_kernel,
        grid=(nsteps,),
        in_specs=[pl.BlockSpec(
            (B, SBLK, D),
            lambda i: (0, jnp.minimum(i, N_NORM - 1), 0))],
        out_specs=[
            pl.BlockSpec((SBLK, D), lambda i: (jnp.minimum(i, N_NORM - 1), 0)),
            pl.BlockSpec((IBLK2,),
                         lambda i: (jnp.maximum(i - (N_NORM + N_RANK), 0),)),
        ],
        out_shape=[jax.ShapeDtypeStruct((S, D), jnp.float32),
                   jax.ShapeDtypeStruct((K,), jnp.int32)],
        scratch_shapes=[pltpu.VMEM((S,), jnp.float32),
                        pltpu.VMEM((S,), jnp.int32)],
    )(hidden_states)
    return hmean, idx


def _norm_kernel(x_ref, imp_ref, hm_ref):
    x = x_ref[...]
    c0 = x[:, :, 0:128]
    acc = c0 * c0
    for c in range(1, 16):
        xc = x[:, :, c * 128:(c + 1) * 128]
        acc = acc + xc * xc
    s2 = acc
    for j in range(1, 16):
        s2 = s2 + pltpu.roll(acc, 128 - 8 * j, axis=2)
    t1 = s2 + pltpu.roll(s2, 124, axis=2)
    t2 = t1 + pltpu.roll(t1, 126, axis=2)
    t3 = t2 + pltpu.roll(t2, 127, axis=2)
    ss = t3[:, :, 0]
    n = jnp.sqrt(ss)
    imp_ref[...] = jnp.mean(n, axis=0)
    hm_ref[...] = jnp.mean(x, axis=0)


def _norm_pass(hidden_states):
    return pl.pallas_call(
        _norm_kernel,
        grid=(S // SBLK,),
        in_specs=[pl.BlockSpec((B, SBLK, D), lambda i: (0, i, 0))],
        out_specs=[pl.BlockSpec((SBLK,), lambda i: (i,)),
                   pl.BlockSpec((SBLK, D), lambda i: (i, 0))],
        out_shape=[jax.ShapeDtypeStruct((S,), jnp.float32),
                   jax.ShapeDtypeStruct((S, D), jnp.float32)],
    )(hidden_states)


def _rank_kernel(imp_full_ref, imp_blk_ref, rank_ref):
    i = pl.program_id(0)
    kt = imp_full_ref[...]
    ks = imp_blk_ref[...]
    ktr = kt[None, :]
    ksc = ks[:, None]
    gt = (ktr > ksc).astype(jnp.int32)
    it = lax.broadcasted_iota(jnp.int32, (RBLK, S), 1)
    isc = i * RBLK + lax.broadcasted_iota(jnp.int32, (RBLK, S), 0)
    tie = ((ktr == ksc) & (it < isc)).astype(jnp.int32)
    rank_ref[...] = jnp.sum(gt + tie, axis=1)


def _rank_pass(imp):
    return pl.pallas_call(
        _rank_kernel,
        grid=(S // RBLK,),
        in_specs=[pl.BlockSpec((S,), lambda i: (0,)),
                  pl.BlockSpec((RBLK,), lambda i: (i,))],
        out_specs=pl.BlockSpec((RBLK,), lambda i: (i,)),
        out_shape=jax.ShapeDtypeStruct((S,), jnp.int32),
    )(imp, imp)


IBLK = 256


def _invert_kernel(rank_ref, idx_ref):
    i = pl.program_id(0)
    r = rank_ref[...]
    rr = r[None, :]
    rows = i * IBLK + lax.broadcasted_iota(jnp.int32, (IBLK, S), 0)
    it = lax.broadcasted_iota(jnp.int32, (IBLK, S), 1)
    sel = jnp.where(rr == rows, it, 0)
    idx_ref[...] = jnp.sum(sel, axis=1)


def _invert_pass(rank):
    return pl.pallas_call(
        _invert_kernel,
        grid=(K // IBLK,),
        in_specs=[pl.BlockSpec((S,), lambda i: (0,))],
        out_specs=pl.BlockSpec((IBLK,), lambda i: (i,)),
        out_shape=jax.ShapeDtypeStruct((K,), jnp.int32),
    )(rank)


def _make_sc_gather():
    info = plsc.get_sparse_core_info()
    nc, ns = info.num_cores, info.num_subcores
    nw = nc * ns
    b_per_w = K // nw
    mesh = plsc.VectorSubcoreMesh(core_axis_name="c", subcore_axis_name="s")

    @functools.partial(
        pl.kernel, mesh=mesh,
        out_type=jax.ShapeDtypeStruct((K, D), jnp.float32),
        scratch_types=[
            pltpu.VMEM((b_per_w,), jnp.int32),
            pltpu.VMEM((b_per_w, D), jnp.float32),
            pltpu.SemaphoreType.DMA,
        ],
    )
    def sc_gather(hmean_hbm, idx_hbm, out_hbm, idx_v, rows_v, sem):
        wid = lax.axis_index("s") * nc + lax.axis_index("c")
        base = wid * b_per_w
        pltpu.sync_copy(idx_hbm.at[pl.ds(base, b_per_w)], idx_v)
        pltpu.async_copy(hmean_hbm.at[idx_v], rows_v, sem).wait()
        pltpu.sync_copy(rows_v, out_hbm.at[pl.ds(base, b_per_w)])

    return sc_gather


def kernel(hidden_states, memory):
    hmean, topk_indices = _fused_pass(hidden_states)
    sc = _make_sc_gather()
    return sc(hmean, topk_indices)
